# baseline (speedup 1.0000x reference)
import functools

import jax
import jax.numpy as jnp
from jax import lax
from jax.experimental import pallas as pl

B = 4
T = 256
DIM = 256
NUM_VARS = 320
GROUPS = 2
VAR_DIM = DIM // GROUPS
GAMMA = 0.25
BT = B * T


def _vq_kernel(x_ref, wt_ref, eg_ref, egt_ref, gnw_ref, gnb_ref,
               out_ref, loss_ref, perp_ref):
    loss_sq = jnp.float32(0.0)
    perp = jnp.float32(0.0)
    iota_v = lax.broadcasted_iota(jnp.int32, (BT, NUM_VARS), 1)
    for g in range(GROUPS):
        xg = x_ref[:, g * VAR_DIM:(g + 1) * VAR_DIM]
        ze = jnp.dot(xg, wt_ref[g], preferred_element_type=jnp.float32)
        slices = []
        for b in range(B):
            blk = ze[b * T:(b + 1) * T, :]
            m = jnp.mean(blk)
            v = jnp.mean((blk - m) * (blk - m))
            slices.append((blk - m) / jnp.sqrt(v + 1e-5))
        zn = jnp.concatenate(slices, axis=0)
        zn = zn * gnw_ref[0, g * VAR_DIM:(g + 1) * VAR_DIM][None, :] \
             + gnb_ref[0, g * VAR_DIM:(g + 1) * VAR_DIM][None, :]
        eg = eg_ref[g]
        egt = egt_ref[g]
        sc = jnp.dot(zn, egt, preferred_element_type=jnp.float32,
                     precision=lax.Precision.HIGHEST)
        e2 = jnp.sum(egt * egt, axis=0, keepdims=True)
        q = 0.5 * e2 - sc
        m1 = jnp.min(q, axis=1, keepdims=True)
        i1 = jnp.min(jnp.where(q == m1, iota_v, NUM_VARS),
                     axis=1, keepdims=True)
        q2 = jnp.where(iota_v == i1, jnp.float32(jnp.inf), q)
        m2 = jnp.min(q2, axis=1, keepdims=True)
        i2 = jnp.min(jnp.where(q2 == m2, iota_v, NUM_VARS),
                     axis=1, keepdims=True)
        oh12 = jnp.concatenate([(iota_v == i1), (iota_v == i2)],
                               axis=0).astype(jnp.float32)
        eab = jnp.dot(oh12, eg, preferred_element_type=jnp.float32,
                      precision=lax.Precision.HIGHEST)
        fab = jnp.concatenate([zn, zn], axis=0) - eab
        d2ab = jnp.sum(fab * fab, axis=1, keepdims=True)
        dab = jnp.sqrt(d2ab)
        ea = eab[:BT, :]
        eb = eab[BT:, :]
        da = dab[:BT, :]
        db = dab[BT:, :]
        win2 = (db < da) | ((db == da) & (i2 < i1))
        zq = jnp.where(win2, eb, ea)
        iw = jnp.where(win2, i2, i1)
        ohw = (iota_v == iw).astype(jnp.float32)
        counts = jnp.sum(ohw, axis=0, keepdims=True)
        p = counts * (1.0 / BT)
        ent = jnp.sum(p * jnp.log(p + 1e-7))
        perp = perp + jnp.exp(-ent)
        dd = zq - zn
        loss_sq = loss_sq + jnp.sum(dd * dd)
        out_ref[:, g * VAR_DIM:(g + 1) * VAR_DIM] = zq
    loss_ref[...] = jnp.reshape(loss_sq * ((1.0 + GAMMA) / (BT * DIM)), (1, 1))
    perp_ref[...] = jnp.reshape(perp, (1, 1))


@functools.partial(jax.jit, static_argnames=("interpret",))
def kernel(x, W_proj, gn_weight, gn_bias, embedding, interpret=False):
    x2 = x.reshape(BT, DIM)
    wt = W_proj.transpose(0, 2, 1)
    eg = embedding.transpose(1, 0, 2)
    egt = embedding.transpose(1, 2, 0)
    gnw = gn_weight.reshape(1, DIM)
    gnb = gn_bias.reshape(1, DIM)
    out, loss, perp = pl.pallas_call(
        _vq_kernel,
        out_shape=[
            jax.ShapeDtypeStruct((BT, DIM), jnp.float32),
            jax.ShapeDtypeStruct((1, 1), jnp.float32),
            jax.ShapeDtypeStruct((1, 1), jnp.float32),
        ],
        interpret=interpret,
    )(x2, wt, eg, egt, gnw, gnb)
    return out.reshape(B, T, DIM), loss[0, 0], perp[0, 0]


# speedup vs baseline: 1.0646x; 1.0646x over previous
import functools

import jax
import jax.numpy as jnp
from jax import lax
from jax.experimental import pallas as pl

B = 4
T = 256
DIM = 256
NUM_VARS = 320
GROUPS = 2
VAR_DIM = DIM // GROUPS
GAMMA = 0.25
BT = B * T


def _vq_kernel(x_ref, wt_ref, eg_ref, egt_ref, gnw_ref, gnb_ref,
               out_ref, loss_ref, perp_ref):
    loss_sq = jnp.float32(0.0)
    perp = jnp.float32(0.0)
    iota_v = lax.broadcasted_iota(jnp.int32, (BT, NUM_VARS), 1)
    for g in range(GROUPS):
        xg = x_ref[:, g * VAR_DIM:(g + 1) * VAR_DIM]
        ze = jnp.dot(xg, wt_ref[g], preferred_element_type=jnp.float32)
        slices = []
        for b in range(B):
            blk = ze[b * T:(b + 1) * T, :]
            m = jnp.mean(blk)
            v = jnp.mean((blk - m) * (blk - m))
            slices.append((blk - m) / jnp.sqrt(v + 1e-5))
        zn = jnp.concatenate(slices, axis=0)
        zn = zn * gnw_ref[0, g * VAR_DIM:(g + 1) * VAR_DIM][None, :] \
             + gnb_ref[0, g * VAR_DIM:(g + 1) * VAR_DIM][None, :]
        eg = eg_ref[g]
        egt = egt_ref[g]
        sc = jnp.dot(zn, egt, preferred_element_type=jnp.float32,
                     precision=lax.Precision.HIGHEST)
        e2 = jnp.sum(egt * egt, axis=0, keepdims=True)
        q = 0.5 * e2 - sc
        m1 = jnp.min(q, axis=1, keepdims=True)
        i1 = jnp.min(jnp.where(q == m1, iota_v, NUM_VARS),
                     axis=1, keepdims=True)
        q2 = jnp.where(iota_v == i1, jnp.float32(jnp.inf), q)
        m2 = jnp.min(q2, axis=1, keepdims=True)
        i2 = jnp.min(jnp.where(q2 == m2, iota_v, NUM_VARS),
                     axis=1, keepdims=True)
        oh1 = (iota_v == i1).astype(jnp.float32)
        oh2 = (iota_v == i2).astype(jnp.float32)
        ea = jnp.dot(oh1, eg, preferred_element_type=jnp.float32,
                     precision=lax.Precision.HIGHEST)
        eb = jnp.dot(oh2, eg, preferred_element_type=jnp.float32,
                     precision=lax.Precision.HIGHEST)
        fa = zn - ea
        fb = zn - eb
        da = jnp.sqrt(jnp.sum(fa * fa, axis=1, keepdims=True))
        db = jnp.sqrt(jnp.sum(fb * fb, axis=1, keepdims=True))
        win2 = (db < da) | ((db == da) & (i2 < i1))
        zq = jnp.where(win2, eb, ea)
        iw = jnp.where(win2, i2, i1)
        ohw = (iota_v == iw).astype(jnp.float32)
        counts = jnp.sum(ohw, axis=0, keepdims=True)
        p = counts * (1.0 / BT)
        ent = jnp.sum(p * jnp.log(p + 1e-7))
        perp = perp + jnp.exp(-ent)
        dd = zq - zn
        loss_sq = loss_sq + jnp.sum(dd * dd)
        out_ref[:, g * VAR_DIM:(g + 1) * VAR_DIM] = zq
    loss_ref[...] = jnp.reshape(loss_sq * ((1.0 + GAMMA) / (BT * DIM)), (1, 1))
    perp_ref[...] = jnp.reshape(perp, (1, 1))


@functools.partial(jax.jit, static_argnames=("interpret",))
def kernel(x, W_proj, gn_weight, gn_bias, embedding, interpret=False):
    x2 = x.reshape(BT, DIM)
    wt = W_proj.transpose(0, 2, 1)
    eg = embedding.transpose(1, 0, 2)
    egt = embedding.transpose(1, 2, 0)
    gnw = gn_weight.reshape(1, DIM)
    gnb = gn_bias.reshape(1, DIM)
    out, loss, perp = pl.pallas_call(
        _vq_kernel,
        out_shape=[
            jax.ShapeDtypeStruct((BT, DIM), jnp.float32),
            jax.ShapeDtypeStruct((1, 1), jnp.float32),
            jax.ShapeDtypeStruct((1, 1), jnp.float32),
        ],
        interpret=interpret,
    )(x2, wt, eg, egt, gnw, gnb)
    return out.reshape(B, T, DIM), loss[0, 0], perp[0, 0]


# speedup vs baseline: 1.0736x; 1.0085x over previous
import functools

import jax
import jax.numpy as jnp
from jax import lax
from jax.experimental import pallas as pl

B = 4
T = 256
DIM = 256
NUM_VARS = 320
GROUPS = 2
VAR_DIM = DIM // GROUPS
GAMMA = 0.25
BT = B * T


def _vq_kernel(x_ref, wt_ref, eg_ref, egt_ref, gnw_ref, gnb_ref,
               out_ref, loss_ref, perp_ref):
    loss_sq = jnp.float32(0.0)
    perp = jnp.float32(0.0)
    iota_v = lax.broadcasted_iota(jnp.int32, (BT, NUM_VARS), 1)
    for g in range(GROUPS):
        xg = x_ref[:, g * VAR_DIM:(g + 1) * VAR_DIM]
        ze = jnp.dot(xg, wt_ref[g], preferred_element_type=jnp.float32)
        slices = []
        for b in range(B):
            blk = ze[b * T:(b + 1) * T, :]
            m = jnp.mean(blk)
            v = jnp.mean((blk - m) * (blk - m))
            slices.append((blk - m) / jnp.sqrt(v + 1e-5))
        zn = jnp.concatenate(slices, axis=0)
        zn = zn * gnw_ref[0, g * VAR_DIM:(g + 1) * VAR_DIM][None, :] \
             + gnb_ref[0, g * VAR_DIM:(g + 1) * VAR_DIM][None, :]
        eg = eg_ref[g]
        egt = egt_ref[g]
        sc = jnp.dot(zn, egt, preferred_element_type=jnp.float32,
                     precision=lax.Precision.HIGHEST)
        e2 = jnp.sum(egt * egt, axis=0, keepdims=True)
        q = 0.5 * e2 - sc
        i1 = jnp.argmin(q, axis=1).astype(jnp.int32).reshape(BT, 1)
        q2 = jnp.where(iota_v == i1, jnp.float32(jnp.inf), q)
        i2 = jnp.argmin(q2, axis=1).astype(jnp.int32).reshape(BT, 1)
        oh1 = (iota_v == i1).astype(jnp.float32)
        oh2 = (iota_v == i2).astype(jnp.float32)
        ea = jnp.dot(oh1, eg, preferred_element_type=jnp.float32,
                     precision=lax.Precision.HIGHEST)
        eb = jnp.dot(oh2, eg, preferred_element_type=jnp.float32,
                     precision=lax.Precision.HIGHEST)
        fa = zn - ea
        fb = zn - eb
        da = jnp.sqrt(jnp.sum(fa * fa, axis=1, keepdims=True))
        db = jnp.sqrt(jnp.sum(fb * fb, axis=1, keepdims=True))
        win2 = (db < da) | ((db == da) & (i2 < i1))
        zq = jnp.where(win2, eb, ea)
        iw = jnp.where(win2, i2, i1)
        ohw = (iota_v == iw).astype(jnp.float32)
        counts = jnp.sum(ohw, axis=0, keepdims=True)
        p = counts * (1.0 / BT)
        ent = jnp.sum(p * jnp.log(p + 1e-7))
        perp = perp + jnp.exp(-ent)
        dd = zq - zn
        loss_sq = loss_sq + jnp.sum(dd * dd)
        out_ref[:, g * VAR_DIM:(g + 1) * VAR_DIM] = zq
    loss_ref[...] = jnp.reshape(loss_sq * ((1.0 + GAMMA) / (BT * DIM)), (1, 1))
    perp_ref[...] = jnp.reshape(perp, (1, 1))


@functools.partial(jax.jit, static_argnames=("interpret",))
def kernel(x, W_proj, gn_weight, gn_bias, embedding, interpret=False):
    x2 = x.reshape(BT, DIM)
    wt = W_proj.transpose(0, 2, 1)
    eg = embedding.transpose(1, 0, 2)
    egt = embedding.transpose(1, 2, 0)
    gnw = gn_weight.reshape(1, DIM)
    gnb = gn_bias.reshape(1, DIM)
    out, loss, perp = pl.pallas_call(
        _vq_kernel,
        out_shape=[
            jax.ShapeDtypeStruct((BT, DIM), jnp.float32),
            jax.ShapeDtypeStruct((1, 1), jnp.float32),
            jax.ShapeDtypeStruct((1, 1), jnp.float32),
        ],
        interpret=interpret,
    )(x2, wt, eg, egt, gnw, gnb)
    return out.reshape(B, T, DIM), loss[0, 0], perp[0, 0]
